# Initial kernel scaffold; baseline (speedup 1.0000x reference)
import functools

import jax
import jax.numpy as jnp
from jax import lax
from jax.experimental import pallas as pl
from jax.experimental.pallas import tpu as pltpu
from jax.experimental.pallas import tpu_sc as plsc

_NC = 2
_NS = 16
_B = 128


def _proj_body(x_ref, wp_ref, bp_ref, wl_ref, bl_ref, wr_ref, br_ref,
               h_ref, xl_ref, xr_ref):
    x = x_ref[...]
    h = jnp.maximum(
        jnp.dot(x, wp_ref[...], preferred_element_type=jnp.float32)
        + bp_ref[...], 0.0)
    h_ref[...] = h
    xl_ref[...] = jnp.dot(h, wl_ref[...],
                          preferred_element_type=jnp.float32) + bl_ref[...]
    xr_ref[...] = jnp.dot(h, wr_ref[...],
                          preferred_element_type=jnp.float32) + br_ref[...]


def _gat_fuse(h, xlv, xrv, pn, ps, att, bo, gg, be, c_per_head):
    u = xlv + xrv
    u = jnp.maximum(u, 0.2 * u)
    lo = u * att
    l0 = jnp.sum(lo[:, :c_per_head], axis=1, keepdims=True)
    l1 = jnp.sum(lo[:, c_per_head:], axis=1, keepdims=True)
    p0 = jnp.exp(l0)
    p1 = jnp.exp(l1)
    num = pn[0] + pn[1]
    s0 = ps[0, :, 0:1] + ps[1, :, 0:1] + p0
    s1 = ps[0, :, 1:2] + ps[1, :, 1:2] + p1
    lane = lax.broadcasted_iota(jnp.int32, num.shape, 1)
    head1 = lane >= c_per_head
    p_self = jnp.where(head1, p1, p0)
    ssum = jnp.where(head1, s1, s0)
    gat = (num + p_self * xlv) / (ssum + 1e-16) + bo
    act = jnp.where(gat > 0, gat, jnp.exp(jnp.minimum(gat, 0.0)) - 1.0)
    hn = h + act
    mu = jnp.mean(hn, axis=1, keepdims=True)
    var = jnp.mean((hn - mu) ** 2, axis=1, keepdims=True)
    return (hn - mu) / jnp.sqrt(var + 1e-5) * gg + be


def _fuse_mid_body(h_ref, xl_ref, xr_ref, pn_ref, ps_ref, att_ref, bo_ref,
                   g_ref, be_ref, wl_ref, bl_ref, wr_ref, br_ref,
                   h1_ref, xl1_ref, xr1_ref, *, c_per_head):
    h1 = _gat_fuse(h_ref[...], xl_ref[...], xr_ref[...], pn_ref[...],
                   ps_ref[...], att_ref[...], bo_ref[...], g_ref[...],
                   be_ref[...], c_per_head)
    h1_ref[...] = h1
    xl1_ref[...] = jnp.dot(h1, wl_ref[...],
                           preferred_element_type=jnp.float32) + bl_ref[...]
    xr1_ref[...] = jnp.dot(h1, wr_ref[...],
                           preferred_element_type=jnp.float32) + br_ref[...]


def _fuse_end_body(h_ref, xl_ref, xr_ref, pn_ref, ps_ref, att_ref, bo_ref,
                   g_ref, be_ref, out_ref, *, c_per_head):
    out_ref[...] = _gat_fuse(h_ref[...], xl_ref[...], xr_ref[...], pn_ref[...],
                             ps_ref[...], att_ref[...], bo_ref[...],
                             g_ref[...], be_ref[...], c_per_head)


@functools.lru_cache(maxsize=None)
def _make_sc_gat(n_nodes, dh, e_real, epw, nchunks):
    mesh = plsc.VectorSubcoreMesh(core_axis_name="c", subcore_axis_name="s")
    rpt = n_nodes // _NS
    nreg = dh // 16
    hreg = nreg // 2

    @functools.partial(
        pl.kernel,
        out_type=(
            jax.ShapeDtypeStruct((_NC, n_nodes, dh), jnp.float32),
            jax.ShapeDtypeStruct((_NC, n_nodes, 16), jnp.float32),
        ),
        mesh=mesh,
        scratch_types=[
            pltpu.VMEM((_B,), jnp.int32),
            pltpu.VMEM((_B,), jnp.int32),
            pltpu.VMEM((_B, dh), jnp.float32),
            pltpu.VMEM((_B, dh), jnp.float32),
            pltpu.VMEM((_B, dh), jnp.float32),
            pltpu.VMEM((_B, 16), jnp.float32),
            pltpu.VMEM((dh,), jnp.float32),
            pltpu.VMEM_SHARED((n_nodes, dh), jnp.float32),
            pltpu.VMEM_SHARED((n_nodes, 16), jnp.float32),
            pltpu.SemaphoreType.DMA,
            pltpu.SemaphoreType.DMA,
        ],
    )
    def sc_gat(xl_hbm, xr_hbm, src_hbm, dst_hbm, att_hbm, outn_hbm, outs_hbm,
               src_v, dst_v, xl_v, xr_v, stn_v, sts_v, att_v,
               accn, accs, sem1, sem2):
        cid = lax.axis_index("c")
        sid = lax.axis_index("s")
        wid = sid * _NC + cid
        pltpu.sync_copy(att_hbm, att_v)
        att_regs = [att_v[pl.ds(k * 16, 16)] for k in range(nreg)]
        zero16 = jnp.zeros((16,), jnp.float32)
        lane = lax.iota(jnp.int32, 16)

        def zrow(i, carry):
            for k in range(nreg):
                stn_v[i, pl.ds(k * 16, 16)] = zero16
            sts_v[i, :] = zero16
            return carry
        lax.fori_loop(0, _B, zrow, 0)
        base_row = sid * rpt
        nfull = rpt // _B
        for k in range(nfull):
            pltpu.sync_copy(stn_v, accn.at[pl.ds(base_row + k * _B, _B)])
            pltpu.sync_copy(sts_v, accs.at[pl.ds(base_row + k * _B, _B)])
        rem = rpt - nfull * _B
        if rem:
            pltpu.sync_copy(stn_v.at[pl.ds(0, rem)],
                            accn.at[pl.ds(base_row + nfull * _B, rem)])
            pltpu.sync_copy(sts_v.at[pl.ds(0, rem)],
                            accs.at[pl.ds(base_row + nfull * _B, rem)])
        plsc.subcore_barrier()

        tile_base = wid * epw

        def chunk(j, carry):
            base = tile_base + j * _B
            pltpu.sync_copy(src_hbm.at[pl.ds(base, _B)], src_v)
            pltpu.sync_copy(dst_hbm.at[pl.ds(base, _B)], dst_v)
            cp1 = pltpu.async_copy(xl_hbm.at[src_v], xl_v, sem1)
            cp2 = pltpu.async_copy(xr_hbm.at[dst_v], xr_v, sem2)
            cp1.wait()
            cp2.wait()

            def edge(e, icarry):
                t = []
                for k in range(nreg):
                    u = (xl_v[e, pl.ds(k * 16, 16)]
                         + xr_v[e, pl.ds(k * 16, 16)])
                    t.append(jnp.maximum(u, 0.2 * u))
                a0 = t[0] * att_regs[0]
                b0 = t[1] * att_regs[1]
                for k in range(2, hreg, 2):
                    a0 = a0 + t[k] * att_regs[k]
                    b0 = b0 + t[k + 1] * att_regs[k + 1]
                a1 = t[hreg] * att_regs[hreg]
                b1 = t[hreg + 1] * att_regs[hreg + 1]
                for k in range(hreg + 2, nreg, 2):
                    a1 = a1 + t[k] * att_regs[k]
                    b1 = b1 + t[k + 1] * att_regs[k + 1]
                l0 = jnp.sum(a0 + b0)
                l1 = jnp.sum(a1 + b1)
                valid = jnp.where(base + e < e_real, 1.0, 0.0)
                p0 = jnp.exp(jnp.full((16,), l0, jnp.float32)) * valid
                p1 = jnp.exp(jnp.full((16,), l1, jnp.float32)) * valid
                for k in range(hreg):
                    stn_v[e, pl.ds(k * 16, 16)] = (
                        p0 * xl_v[e, pl.ds(k * 16, 16)])
                for k in range(hreg, nreg):
                    stn_v[e, pl.ds(k * 16, 16)] = (
                        p1 * xl_v[e, pl.ds(k * 16, 16)])
                sts_v[e, :] = jnp.where(lane == 0, p0,
                                        jnp.where(lane == 1, p1, 0.0))
                return icarry
            lax.fori_loop(0, _B, edge, 0)
            pltpu.sync_copy(stn_v, accn.at[dst_v], add=True)
            pltpu.sync_copy(sts_v, accs.at[dst_v], add=True)
            return carry
        lax.fori_loop(0, nchunks, chunk, 0)
        plsc.subcore_barrier()

        pltpu.sync_copy(accn.at[pl.ds(base_row, rpt)],
                        outn_hbm.at[cid, pl.ds(base_row, rpt)])
        pltpu.sync_copy(accs.at[pl.ds(base_row, rpt)],
                        outs_hbm.at[cid, pl.ds(base_row, rpt)])

    return sc_gat


def _proj3(x, wp, bp, wl, bl, wr, br, br_rows):
    n, d = x.shape
    dh = wl.shape[1]
    rb = lambda i: (i, 0)
    zb = lambda i: (0, 0)
    return pl.pallas_call(
        _proj_body,
        grid=(n // br_rows,),
        in_specs=[pl.BlockSpec((br_rows, d), rb),
                  pl.BlockSpec((d, d), zb), pl.BlockSpec((1, d), zb),
                  pl.BlockSpec((d, dh), zb), pl.BlockSpec((1, dh), zb),
                  pl.BlockSpec((d, dh), zb), pl.BlockSpec((1, dh), zb)],
        out_specs=[pl.BlockSpec((br_rows, dh), rb)] * 3,
        out_shape=[jax.ShapeDtypeStruct((n, dh), jnp.float32)] * 3,
    )(x, wp, bp.reshape(1, -1), wl, bl.reshape(1, -1), wr, br.reshape(1, -1))


def _fuse_mid(h, xl, xr, pn, ps, att_f, bo, gg, be, wl, bl, wr, br, br_rows,
              c_per_head):
    n, d = h.shape
    dh = wl.shape[1]
    rb = lambda i: (i, 0)
    zb = lambda i: (0, 0)
    rb3 = lambda i: (0, i, 0)
    return pl.pallas_call(
        functools.partial(_fuse_mid_body, c_per_head=c_per_head),
        grid=(n // br_rows,),
        in_specs=[pl.BlockSpec((br_rows, d), rb),
                  pl.BlockSpec((br_rows, dh), rb),
                  pl.BlockSpec((br_rows, dh), rb),
                  pl.BlockSpec((_NC, br_rows, dh), rb3),
                  pl.BlockSpec((_NC, br_rows, 16), rb3),
                  pl.BlockSpec((1, dh), zb), pl.BlockSpec((1, dh), zb),
                  pl.BlockSpec((1, d), zb), pl.BlockSpec((1, d), zb),
                  pl.BlockSpec((d, dh), zb), pl.BlockSpec((1, dh), zb),
                  pl.BlockSpec((d, dh), zb), pl.BlockSpec((1, dh), zb)],
        out_specs=[pl.BlockSpec((br_rows, d), rb),
                   pl.BlockSpec((br_rows, dh), rb),
                   pl.BlockSpec((br_rows, dh), rb)],
        out_shape=[jax.ShapeDtypeStruct((n, d), jnp.float32),
                   jax.ShapeDtypeStruct((n, dh), jnp.float32),
                   jax.ShapeDtypeStruct((n, dh), jnp.float32)],
    )(h, xl, xr, pn, ps, att_f.reshape(1, -1), bo.reshape(1, -1),
      gg.reshape(1, -1), be.reshape(1, -1), wl, bl.reshape(1, -1),
      wr, br.reshape(1, -1))


def _fuse_end(h, xl, xr, pn, ps, att_f, bo, gg, be, br_rows, c_per_head):
    n, d = h.shape
    dh = xl.shape[1]
    rb = lambda i: (i, 0)
    zb = lambda i: (0, 0)
    rb3 = lambda i: (0, i, 0)
    return pl.pallas_call(
        functools.partial(_fuse_end_body, c_per_head=c_per_head),
        grid=(n // br_rows,),
        in_specs=[pl.BlockSpec((br_rows, d), rb),
                  pl.BlockSpec((br_rows, dh), rb),
                  pl.BlockSpec((br_rows, dh), rb),
                  pl.BlockSpec((_NC, br_rows, dh), rb3),
                  pl.BlockSpec((_NC, br_rows, 16), rb3),
                  pl.BlockSpec((1, dh), zb), pl.BlockSpec((1, dh), zb),
                  pl.BlockSpec((1, d), zb), pl.BlockSpec((1, d), zb)],
        out_specs=pl.BlockSpec((br_rows, d), rb),
        out_shape=jax.ShapeDtypeStruct((n, d), jnp.float32),
    )(h, xl, xr, pn, ps, att_f.reshape(1, -1), bo.reshape(1, -1),
      gg.reshape(1, -1), be.reshape(1, -1))


def kernel(x, edge_index, Wp, bp, Wl0, bl0, Wr0, br0, att0, bo0, g0, be0,
           Wl1, bl1, Wr1, br1, att1, bo1, g1, be1):
    n, d = x.shape
    e = edge_index.shape[1]
    dh = Wl0.shape[1]
    c_per_head = att0.shape[1]
    nw = _NC * _NS
    nchunks = -(-e // (_B * nw))
    epw = _B * nchunks
    e_pad = epw * nw
    src = edge_index[0]
    dst = edge_index[1]
    if e_pad > e:
        zpad = jnp.zeros((e_pad - e,), jnp.int32)
        src = jnp.concatenate([src, zpad])
        dst = jnp.concatenate([dst, zpad])
    br_rows = 1000 if n % 1000 == 0 else 8 * (n // 8)

    sc_gat = _make_sc_gat(n, dh, e, epw, nchunks)

    h, xl0a, xr0a = _proj3(x, Wp, bp, Wl0, bl0, Wr0, br0, br_rows)
    pn0, ps0 = sc_gat(xl0a, xr0a, src, dst, att0.reshape(-1))
    h1, xl1a, xr1a = _fuse_mid(h, xl0a, xr0a, pn0, ps0, att0.reshape(-1),
                               bo0, g0, be0, Wl1, bl1, Wr1, br1, br_rows,
                               c_per_head)
    pn1, ps1 = sc_gat(xl1a, xr1a, src, dst, att1.reshape(-1))
    return _fuse_end(h1, xl1a, xr1a, pn1, ps1, att1.reshape(-1), bo1, g1, be1,
                     br_rows, c_per_head)


# speedup vs baseline: 30.6626x; 30.6626x over previous
import functools

import jax
import jax.numpy as jnp
from jax import lax
from jax.experimental import pallas as pl
from jax.experimental.pallas import tpu as pltpu
from jax.experimental.pallas import tpu_sc as plsc

_NC = 2
_NS = 16
_B = 48


def _proj_body(x_ref, wp_ref, bp_ref, wl_ref, bl_ref, wr_ref, br_ref,
               h_ref, xl_ref, xr_ref):
    x = x_ref[...]
    h = jnp.maximum(
        jnp.dot(x, wp_ref[...], preferred_element_type=jnp.float32)
        + bp_ref[...], 0.0)
    h_ref[...] = h
    xl_ref[...] = jnp.dot(h, wl_ref[...],
                          preferred_element_type=jnp.float32) + bl_ref[...]
    xr_ref[...] = jnp.dot(h, wr_ref[...],
                          preferred_element_type=jnp.float32) + br_ref[...]


def _gat_fuse(h, xlv, xrv, pn, ps, att, bo, gg, be, c_per_head):
    u = xlv + xrv
    u = jnp.maximum(u, 0.2 * u)
    lo = u * att
    l0 = jnp.sum(lo[:, :c_per_head], axis=1, keepdims=True)
    l1 = jnp.sum(lo[:, c_per_head:], axis=1, keepdims=True)
    p0 = jnp.exp(l0)
    p1 = jnp.exp(l1)
    num = pn[0] + pn[1]
    s0 = ps[:, 0:1] + p0
    s1 = ps[:, 1:2] + p1
    lane = lax.broadcasted_iota(jnp.int32, num.shape, 1)
    head1 = lane >= c_per_head
    p_self = jnp.where(head1, p1, p0)
    ssum = jnp.where(head1, s1, s0)
    gat = (num + p_self * xlv) / (ssum + 1e-16) + bo
    act = jnp.where(gat > 0, gat, jnp.exp(jnp.minimum(gat, 0.0)) - 1.0)
    hn = h + act
    mu = jnp.mean(hn, axis=1, keepdims=True)
    var = jnp.mean((hn - mu) ** 2, axis=1, keepdims=True)
    return (hn - mu) / jnp.sqrt(var + 1e-5) * gg + be


def _fuse_mid_body(h_ref, xl_ref, xr_ref, pn_ref, ps_ref, att_ref, bo_ref,
                   g_ref, be_ref, wl_ref, bl_ref, wr_ref, br_ref,
                   h1_ref, xl1_ref, xr1_ref, *, c_per_head):
    h1 = _gat_fuse(h_ref[...], xl_ref[...], xr_ref[...], pn_ref[...],
                   ps_ref[...], att_ref[...], bo_ref[...], g_ref[...],
                   be_ref[...], c_per_head)
    h1_ref[...] = h1
    xl1_ref[...] = jnp.dot(h1, wl_ref[...],
                           preferred_element_type=jnp.float32) + bl_ref[...]
    xr1_ref[...] = jnp.dot(h1, wr_ref[...],
                           preferred_element_type=jnp.float32) + br_ref[...]


def _fuse_end_body(h_ref, xl_ref, xr_ref, pn_ref, ps_ref, att_ref, bo_ref,
                   g_ref, be_ref, out_ref, *, c_per_head):
    out_ref[...] = _gat_fuse(h_ref[...], xl_ref[...], xr_ref[...], pn_ref[...],
                             ps_ref[...], att_ref[...], bo_ref[...],
                             g_ref[...], be_ref[...], c_per_head)


@functools.lru_cache(maxsize=None)
def _make_sc_gat(n_pad, dh, e_real, epw, nchunks):
    mesh = plsc.VectorSubcoreMesh(core_axis_name="c", subcore_axis_name="s")
    rpt = n_pad // _NS
    nreg = dh // 16
    hreg = nreg // 2

    nw = _NC * _NS

    @functools.partial(
        pl.kernel,
        out_type=(
            jax.ShapeDtypeStruct((_NC, n_pad, dh), jnp.float32),
            jax.ShapeDtypeStruct((nw, 2 * n_pad), jnp.float32),
        ),
        mesh=mesh,
        scratch_types=[
            pltpu.VMEM((_B,), jnp.int32),
            pltpu.VMEM((_B,), jnp.int32),
            pltpu.VMEM((_B, dh), jnp.float32),
            pltpu.VMEM((_B, dh), jnp.float32),
            pltpu.VMEM((dh,), jnp.float32),
            pltpu.VMEM((2 * n_pad,), jnp.float32),
            pltpu.VMEM_SHARED((n_pad, dh), jnp.float32),
            pltpu.SemaphoreType.DMA,
            pltpu.SemaphoreType.DMA,
        ],
        compiler_params=pltpu.CompilerParams(needs_layout_passes=False),
    )
    def sc_gat(xl_hbm, xr_hbm, src_hbm, dst_hbm, att_hbm, outn_hbm, outd_hbm,
               src_v, dst_v, xl_v, xr_v, att_v, accd, accn, sem1, sem2):
        cid = lax.axis_index("c")
        sid = lax.axis_index("s")
        wid = sid * _NC + cid
        pltpu.sync_copy(att_hbm, att_v)
        att_regs = [att_v[pl.ds(k * 16, 16)] for k in range(nreg)]
        lane = lax.iota(jnp.int32, 16)
        zero16 = jnp.where(lane >= 0, 0.0, 1.0)
        col16 = jnp.minimum(lane, 1)
        dmask = lane < 2
        perms = [(lane + sh) & 15 for sh in (8, 4, 2, 1)]

        def lane_sum(v):
            for perm in perms:
                v = v + v.at[perm].get(mode="promise_in_bounds")
            return v

        def zd(i, carry):
            accd[pl.ds(i * 16, 16)] = zero16
            return carry
        lax.fori_loop(0, 2 * n_pad // 16, zd, 0)

        def zrow(i, carry):
            for k in range(nreg):
                xl_v[i, pl.ds(k * 16, 16)] = zero16
            return carry
        lax.fori_loop(0, _B, zrow, 0)
        base_row = sid * rpt
        nfull = rpt // _B
        for k in range(nfull):
            pltpu.sync_copy(xl_v, accn.at[pl.ds(base_row + k * _B, _B)])
        rem = rpt - nfull * _B
        if rem:
            pltpu.sync_copy(xl_v.at[pl.ds(0, rem)],
                            accn.at[pl.ds(base_row + nfull * _B, rem)])
        plsc.subcore_barrier()

        tile_base = wid * epw

        def chunk(j, carry):
            base = tile_base + j * _B
            pltpu.sync_copy(src_hbm.at[pl.ds(base, _B)], src_v)
            pltpu.sync_copy(dst_hbm.at[pl.ds(base, _B)], dst_v)
            cp1 = pltpu.async_copy(xl_hbm.at[src_v], xl_v, sem1)
            cp2 = pltpu.async_copy(xr_hbm.at[dst_v], xr_v, sem2)
            cp1.wait()
            cp2.wait()

            def edge_grp(g, icarry):
                dvec2 = 2 * dst_v[pl.ds(g * 16, 16)]
                for e16 in range(16):
                    e = g * 16 + e16
                    t = []
                    for k in range(nreg):
                        u = (xl_v[e, pl.ds(k * 16, 16)]
                             + xr_v[e, pl.ds(k * 16, 16)])
                        t.append(jnp.maximum(u, 0.2 * u))
                    a0 = t[0] * att_regs[0]
                    b0 = t[1] * att_regs[1]
                    for k in range(2, hreg, 2):
                        a0 = a0 + t[k] * att_regs[k]
                        b0 = b0 + t[k + 1] * att_regs[k + 1]
                    a1 = t[hreg] * att_regs[hreg]
                    b1 = t[hreg + 1] * att_regs[hreg + 1]
                    for k in range(hreg + 2, nreg, 2):
                        a1 = a1 + t[k] * att_regs[k]
                        b1 = b1 + t[k + 1] * att_regs[k + 1]
                    l0 = lane_sum(a0 + b0)
                    l1 = lane_sum(a1 + b1)
                    valid = jnp.where(base + e < e_real, 1.0, 0.0)
                    p0 = jnp.exp(l0) * valid
                    p1 = jnp.exp(l1) * valid
                    for k in range(hreg):
                        xl_v[e, pl.ds(k * 16, 16)] = (
                            p0 * xl_v[e, pl.ds(k * 16, 16)])
                    for k in range(hreg, nreg):
                        xl_v[e, pl.ds(k * 16, 16)] = (
                            p1 * xl_v[e, pl.ds(k * 16, 16)])
                    dsplat = dvec2.at[lane * 0 + e16].get(
                        mode="promise_in_bounds")
                    plsc.addupdate_scatter(
                        accd, [dsplat + col16],
                        jnp.where(lane == 0, p0,
                                  jnp.where(lane == 1, p1, 0.0)),
                        mask=dmask)
                return icarry
            lax.fori_loop(0, _B // 16, edge_grp, 0)
            pltpu.sync_copy(xl_v, accn.at[dst_v], add=True)
            return carry
        lax.fori_loop(0, nchunks, chunk, 0)
        plsc.subcore_barrier()

        pltpu.sync_copy(accn.at[pl.ds(base_row, rpt)],
                        outn_hbm.at[cid, pl.ds(base_row, rpt)])
        pltpu.sync_copy(accd, outd_hbm.at[wid])

    return sc_gat


def _dreduce_body(pd_ref, out_ref):
    out_ref[...] = jnp.sum(pd_ref[...], axis=0, keepdims=True)


def _dreduce(pd_flat):
    nwrk, length = pd_flat.shape
    return pl.pallas_call(
        _dreduce_body,
        in_specs=[pl.BlockSpec((nwrk, length), lambda: (0, 0))],
        out_specs=pl.BlockSpec((1, length), lambda: (0, 0)),
        out_shape=jax.ShapeDtypeStruct((1, length), jnp.float32),
    )(pd_flat)


def _proj3(x, wp, bp, wl, bl, wr, br, br_rows):
    n, d = x.shape
    dh = wl.shape[1]
    rb = lambda i: (i, 0)
    zb = lambda i: (0, 0)
    return pl.pallas_call(
        _proj_body,
        grid=(n // br_rows,),
        in_specs=[pl.BlockSpec((br_rows, d), rb),
                  pl.BlockSpec((d, d), zb), pl.BlockSpec((1, d), zb),
                  pl.BlockSpec((d, dh), zb), pl.BlockSpec((1, dh), zb),
                  pl.BlockSpec((d, dh), zb), pl.BlockSpec((1, dh), zb)],
        out_specs=[pl.BlockSpec((br_rows, dh), rb)] * 3,
        out_shape=[jax.ShapeDtypeStruct((n, dh), jnp.float32)] * 3,
    )(x, wp, bp.reshape(1, -1), wl, bl.reshape(1, -1), wr, br.reshape(1, -1))


def _fuse_mid(h, xl, xr, pn, ps, att_f, bo, gg, be, wl, bl, wr, br, br_rows,
              c_per_head):
    n, d = h.shape
    dh = wl.shape[1]
    rb = lambda i: (i, 0)
    zb = lambda i: (0, 0)
    rb3 = lambda i: (0, i, 0)
    return pl.pallas_call(
        functools.partial(_fuse_mid_body, c_per_head=c_per_head),
        grid=(n // br_rows,),
        in_specs=[pl.BlockSpec((br_rows, d), rb),
                  pl.BlockSpec((br_rows, dh), rb),
                  pl.BlockSpec((br_rows, dh), rb),
                  pl.BlockSpec((_NC, br_rows, dh), rb3),
                  pl.BlockSpec((br_rows, 2), rb),
                  pl.BlockSpec((1, dh), zb), pl.BlockSpec((1, dh), zb),
                  pl.BlockSpec((1, d), zb), pl.BlockSpec((1, d), zb),
                  pl.BlockSpec((d, dh), zb), pl.BlockSpec((1, dh), zb),
                  pl.BlockSpec((d, dh), zb), pl.BlockSpec((1, dh), zb)],
        out_specs=[pl.BlockSpec((br_rows, d), rb),
                   pl.BlockSpec((br_rows, dh), rb),
                   pl.BlockSpec((br_rows, dh), rb)],
        out_shape=[jax.ShapeDtypeStruct((n, d), jnp.float32),
                   jax.ShapeDtypeStruct((n, dh), jnp.float32),
                   jax.ShapeDtypeStruct((n, dh), jnp.float32)],
    )(h, xl, xr, pn, ps, att_f.reshape(1, -1), bo.reshape(1, -1),
      gg.reshape(1, -1), be.reshape(1, -1), wl, bl.reshape(1, -1),
      wr, br.reshape(1, -1))


def _fuse_end(h, xl, xr, pn, ps, att_f, bo, gg, be, br_rows, c_per_head):
    n, d = h.shape
    dh = xl.shape[1]
    rb = lambda i: (i, 0)
    zb = lambda i: (0, 0)
    rb3 = lambda i: (0, i, 0)
    return pl.pallas_call(
        functools.partial(_fuse_end_body, c_per_head=c_per_head),
        grid=(n // br_rows,),
        in_specs=[pl.BlockSpec((br_rows, d), rb),
                  pl.BlockSpec((br_rows, dh), rb),
                  pl.BlockSpec((br_rows, dh), rb),
                  pl.BlockSpec((_NC, br_rows, dh), rb3),
                  pl.BlockSpec((br_rows, 2), rb),
                  pl.BlockSpec((1, dh), zb), pl.BlockSpec((1, dh), zb),
                  pl.BlockSpec((1, d), zb), pl.BlockSpec((1, d), zb)],
        out_specs=pl.BlockSpec((br_rows, d), rb),
        out_shape=jax.ShapeDtypeStruct((n, d), jnp.float32),
    )(h, xl, xr, pn, ps, att_f.reshape(1, -1), bo.reshape(1, -1),
      gg.reshape(1, -1), be.reshape(1, -1))


def kernel(x, edge_index, Wp, bp, Wl0, bl0, Wr0, br0, att0, bo0, g0, be0,
           Wl1, bl1, Wr1, br1, att1, bo1, g1, be1):
    n, d = x.shape
    e = edge_index.shape[1]
    dh = Wl0.shape[1]
    c_per_head = att0.shape[1]
    nw = _NC * _NS
    nchunks = -(-e // (_B * nw))
    epw = _B * nchunks
    e_pad = epw * nw
    src = edge_index[0]
    dst = edge_index[1]
    if e_pad > e:
        zpad = jnp.zeros((e_pad - e,), jnp.int32)
        src = jnp.concatenate([src, zpad])
        dst = jnp.concatenate([dst, zpad])
    br_rows = 1000 if n % 1000 == 0 else 8 * (n // 8)
    n_pad = 128 * -(-n // 128)

    sc_gat = _make_sc_gat(n_pad, dh, e, epw, nchunks)

    h, xl0a, xr0a = _proj3(x, Wp, bp, Wl0, bl0, Wr0, br0, br_rows)
    pn0, pd0 = sc_gat(xl0a, xr0a, src, dst, att0.reshape(-1))
    pd0 = _dreduce(pd0).reshape(n_pad, 2)
    h1, xl1a, xr1a = _fuse_mid(h, xl0a, xr0a, pn0, pd0, att0.reshape(-1),
                               bo0, g0, be0, Wl1, bl1, Wr1, br1, br_rows,
                               c_per_head)
    pn1, pd1 = sc_gat(xl1a, xr1a, src, dst, att1.reshape(-1))
    pd1 = _dreduce(pd1).reshape(n_pad, 2)
    return _fuse_end(h1, xl1a, xr1a, pn1, pd1, att1.reshape(-1), bo1, g1, be1,
                     br_rows, c_per_head)
